# baseline (speedup 1.0000x reference)
import functools

import jax
import jax.numpy as jnp
from jax import lax
from jax.experimental import pallas as pl
from jax.experimental.pallas import tpu as pltpu
from jax.experimental.pallas import tpu_sc as plsc

VOCAB = 1000000
EMBED = 64
BATCH = 4096
HIST = 50

NC = 2
NS = 16
NW = NC * NS
TOTAL = BATCH * HIST
PER_W = TOTAL // NW
CHUNK = 128
NCHUNK = PER_W // CHUNK
NBUF = 5

TBLK = 8192
TGRID = -(-VOCAB // TBLK)
HB = TBLK // 2
TAIL = VOCAB % TBLK
FULL_END = VOCAB - TAIL

_mesh = plsc.VectorSubcoreMesh(core_axis_name="c", subcore_axis_name="s")


def _pack_body(wt_ref, out_ref):
    g = pl.program_id(0)
    x = wt_ref[...]
    xt = jnp.swapaxes(x, 0, 1)

    @pl.when(g < TGRID - 1)
    def _():
        out_ref[...] = jnp.concatenate([xt[:HB], xt[HB:]], axis=1)

    @pl.when(g == TGRID - 1)
    def _():
        out_ref[0 : TAIL // 2, :] = jnp.concatenate(
            [xt[0 : TAIL // 2], xt[TAIL // 2 : TAIL]], axis=1
        )


@jax.jit
def _pack_table(wt):
    return pl.pallas_call(
        _pack_body,
        grid=(TGRID,),
        in_specs=[pl.BlockSpec((EMBED, TBLK), lambda g: (0, g))],
        out_specs=pl.BlockSpec((TBLK // 2, 2 * EMBED), lambda g: (g, 0)),
        out_shape=jax.ShapeDtypeStruct((VOCAB // 2, 2 * EMBED), jnp.float32),
    )(wt)


EG = EMBED // 8
BT = BATCH // CHUNK


@functools.partial(
    pl.kernel,
    mesh=_mesh,
    out_type=jax.ShapeDtypeStruct((HIST, EG, BT, 8, CHUNK), jnp.float32),
    scratch_types=[
        pltpu.VMEM((NCHUNK, CHUNK), jnp.int32),
        [pltpu.VMEM((CHUNK, EMBED), jnp.float32) for _ in range(NBUF)],
        [pltpu.VMEM((EG, 8, CHUNK), jnp.float32) for _ in range(NBUF)],
        [pltpu.SemaphoreType.DMA for _ in range(NBUF)],
        [pltpu.SemaphoreType.DMA for _ in range(NBUF)],
    ],
    compiler_params=pltpu.CompilerParams(use_tc_tiling_on_sc=False, needs_layout_passes=False),
)
def _gather(table_hbm, idx_hbm, out_hbm, idx_v, rows, rowsT, sem_g, sem_w):
    wid = lax.axis_index("s") * NC + lax.axis_index("c")
    base_u = wid * NCHUNK
    pltpu.sync_copy(idx_hbm.at[wid], idx_v)

    def start_gather(j, b):
        pltpu.async_copy(table_hbm.at[idx_v.at[j]], rows[b], sem_g[b])

    def transpose_chunk(b):
        base_lanes = lax.iota(jnp.int32, 16)
        for e in range(EMBED):
            cols = jnp.full((16,), e, jnp.int32)
            for l0 in range(0, CHUNK, 16):
                vals = plsc.load_gather(rows[b], [base_lanes + l0, cols])
                rowsT[b][e // 8, e % 8, pl.ds(l0, 16)] = vals

    def writeback(j, b):
        u = base_u + j
        h = u // BT
        c = u % BT
        pltpu.async_copy(rowsT[b], out_hbm.at[h, :, c], sem_w[b])

    for j in range(NBUF):
        start_gather(j, j)

    def group(g, carry):
        for b in range(NBUF):
            j = g * NBUF + b
            pltpu.make_async_copy(table_hbm.at[idx_v.at[j]], rows[b], sem_g[b]).wait()

            @pl.when(g > 0)
            def _():
                pltpu.make_async_copy(
                    rowsT[b], out_hbm.at[0, :, 0], sem_w[b]
                ).wait()

            transpose_chunk(b)
            writeback(j, b)

            @pl.when(j + NBUF < NCHUNK)
            def _():
                start_gather(j + NBUF, b)
        return carry

    lax.fori_loop(0, NCHUNK // NBUF, group, 0)
    for b in range(NBUF):
        pltpu.make_async_copy(rowsT[b], out_hbm.at[0, :, 0], sem_w[b]).wait()


def kernel(inputs, weight):
    packed = _pack_table(weight.T)
    table = packed.reshape(VOCAB, EMBED)
    v = inputs.astype(jnp.int32)
    l = v % TBLK
    u_full = (v - l) + jnp.where(l < HB, 2 * l, 2 * l - (TBLK - 1))
    t = v - FULL_END
    u_tail = FULL_END + jnp.where(t < TAIL // 2, 2 * t, 2 * t - (TAIL - 1))
    u = jnp.where(v < FULL_END, u_full, u_tail)
    idx = u.T.reshape(NW, NCHUNK, CHUNK)
    out5 = _gather(table, idx)
    return out5.transpose(2, 4, 0, 1, 3).reshape(BATCH, HIST, EMBED)


# speedup vs baseline: 1.6099x; 1.6099x over previous
import functools

import jax
import jax.numpy as jnp
from jax import lax
from jax.experimental import pallas as pl
from jax.experimental.pallas import tpu as pltpu
from jax.experimental.pallas import tpu_sc as plsc

VOCAB = 1000000
EMBED = 64
BATCH = 4096
HIST = 50

NC = 2
NS = 16
NW = NC * NS
TOTAL = BATCH * HIST
PER_W = TOTAL // NW
CHUNK = 128
NCHUNK = PER_W // CHUNK
NBUF = 8
LOOKAHEAD = 4

TBLK = 16384
TGRID = -(-VOCAB // TBLK)
HB = TBLK // 2
TAIL = VOCAB % TBLK
FULL_END = VOCAB - TAIL

_mesh = plsc.VectorSubcoreMesh(core_axis_name="c", subcore_axis_name="s")


def _pack_body(wt_ref, out_ref):
    g = pl.program_id(0)
    x = wt_ref[...]
    xt = jnp.swapaxes(x, 0, 1)

    @pl.when(g < TGRID - 1)
    def _():
        out_ref[...] = jnp.concatenate([xt[:HB], xt[HB:]], axis=1)

    @pl.when(g == TGRID - 1)
    def _():
        out_ref[0 : TAIL // 2, :] = jnp.concatenate(
            [xt[0 : TAIL // 2], xt[TAIL // 2 : TAIL]], axis=1
        )


@jax.jit
def _pack_table(wt):
    return pl.pallas_call(
        _pack_body,
        grid=(TGRID,),
        in_specs=[pl.BlockSpec((EMBED, TBLK), lambda g: (0, g))],
        out_specs=pl.BlockSpec((TBLK // 2, 2 * EMBED), lambda g: (g, 0)),
        out_shape=jax.ShapeDtypeStruct((VOCAB // 2, 2 * EMBED), jnp.float32),
    )(wt)


@functools.partial(
    pl.kernel,
    mesh=_mesh,
    out_type=jax.ShapeDtypeStruct((TOTAL, EMBED), jnp.float32),
    scratch_types=[
        pltpu.VMEM((NCHUNK, CHUNK), jnp.int32),
        [pltpu.VMEM((CHUNK, EMBED), jnp.float32) for _ in range(NBUF)],
        [pltpu.SemaphoreType.DMA for _ in range(NBUF)],
        [pltpu.SemaphoreType.DMA for _ in range(NBUF)],
    ],
    compiler_params=pltpu.CompilerParams(use_tc_tiling_on_sc=False),
)
def _gather(table_hbm, idx_hbm, out_hbm, idx_v, rows, sem_g, sem_w):
    wid = lax.axis_index("s") * NC + lax.axis_index("c")
    base = wid * PER_W
    pltpu.sync_copy(idx_hbm.at[wid], idx_v)

    copies_g = [None] * NBUF
    copies_w = [None] * NBUF

    def start_gather(j):
        b = j % NBUF
        copies_g[b] = pltpu.async_copy(table_hbm.at[idx_v.at[j]], rows[b], sem_g[b])

    for j in range(LOOKAHEAD):
        start_gather(j)

    for j in range(NCHUNK):
        b = j % NBUF
        nj = j + LOOKAHEAD
        if nj < NCHUNK:
            bn = nj % NBUF
            if copies_w[bn] is not None:
                copies_w[bn].wait()
            start_gather(nj)
        copies_g[b].wait()
        copies_w[b] = pltpu.async_copy(
            rows[b], out_hbm.at[pl.ds(base + j * CHUNK, CHUNK)], sem_w[b]
        )
    for b in range(NBUF):
        if copies_w[b] is not None:
            copies_w[b].wait()


def kernel(inputs, weight):
    packed = _pack_table(weight.T)
    table = packed.reshape(VOCAB, EMBED)
    v = inputs.astype(jnp.int32)
    l = v % TBLK
    u_full = (v - l) + jnp.where(l < HB, 2 * l, 2 * l - (TBLK - 1))
    t = v - FULL_END
    u_tail = FULL_END + jnp.where(t < TAIL // 2, 2 * t, 2 * t - (TAIL - 1))
    u = jnp.where(v < FULL_END, u_full, u_tail)
    idx = u.reshape(NW, NCHUNK, CHUNK)
    out = _gather(table, idx)
    return out.reshape(BATCH, HIST, EMBED)


# speedup vs baseline: 1.6170x; 1.0044x over previous
import functools

import jax
import jax.numpy as jnp
from jax import lax
from jax.experimental import pallas as pl
from jax.experimental.pallas import tpu as pltpu
from jax.experimental.pallas import tpu_sc as plsc

VOCAB = 1000000
EMBED = 64
BATCH = 4096
HIST = 50

NC = 2
NS = 16
NW = NC * NS
TOTAL = BATCH * HIST
PER_W = TOTAL // NW
CHUNK = 128
NCHUNK = PER_W // CHUNK
NBUF = 8
LOOKAHEAD = 4

TBLK = 16384
TGRID = -(-VOCAB // TBLK)
HB = TBLK // 2
TAIL = VOCAB % TBLK
FULL_END = VOCAB - TAIL

_mesh = plsc.VectorSubcoreMesh(core_axis_name="c", subcore_axis_name="s")


def _pack_body(wt_ref, out_ref):
    g = pl.program_id(0)
    x = wt_ref[...]
    xt = jnp.swapaxes(x, 0, 1)

    @pl.when(g < TGRID - 1)
    def _():
        out_ref[...] = jnp.concatenate([xt[:HB], xt[HB:]], axis=1)

    @pl.when(g == TGRID - 1)
    def _():
        out_ref[0 : TAIL // 2, :] = jnp.concatenate(
            [xt[0 : TAIL // 2], xt[TAIL // 2 : TAIL]], axis=1
        )


@jax.jit
def _pack_table(wt):
    return pl.pallas_call(
        _pack_body,
        grid=(TGRID,),
        in_specs=[pl.BlockSpec((EMBED, TBLK), lambda g: (0, g))],
        out_specs=pl.BlockSpec((TBLK // 2, 2 * EMBED), lambda g: (g, 0)),
        out_shape=jax.ShapeDtypeStruct((VOCAB // 2, 2 * EMBED), jnp.float32),
    )(wt)


def _fmt_body(in_ref, out_ref):
    x = in_ref[0]
    ya = jnp.swapaxes(x[:, :EMBED], 0, 1)
    yb = jnp.swapaxes(x[:, EMBED:], 0, 1)
    out_ref[0] = jnp.concatenate([ya, yb], axis=1)


@jax.jit
def _fmt_out(rows2):
    return pl.pallas_call(
        _fmt_body,
        grid=(HIST,),
        in_specs=[pl.BlockSpec((1, BATCH // 2, 2 * EMBED), lambda g: (g, 0, 0))],
        out_specs=pl.BlockSpec((1, EMBED, BATCH), lambda g: (g, 0, 0)),
        out_shape=jax.ShapeDtypeStruct((HIST, EMBED, BATCH), jnp.float32),
    )(rows2)


@functools.partial(
    pl.kernel,
    mesh=_mesh,
    out_type=jax.ShapeDtypeStruct((TOTAL, EMBED), jnp.float32),
    scratch_types=[
        pltpu.VMEM((NCHUNK, CHUNK), jnp.int32),
        [pltpu.VMEM((CHUNK, EMBED), jnp.float32) for _ in range(NBUF)],
        [pltpu.SemaphoreType.DMA for _ in range(NBUF)],
        [pltpu.SemaphoreType.DMA for _ in range(NBUF)],
    ],
    compiler_params=pltpu.CompilerParams(use_tc_tiling_on_sc=False),
)
def _gather(table_hbm, idx_hbm, out_hbm, idx_v, rows, sem_g, sem_w):
    wid = lax.axis_index("s") * NC + lax.axis_index("c")
    base = wid * PER_W
    pltpu.sync_copy(idx_hbm.at[wid], idx_v)

    copies_g = [None] * NBUF
    copies_w = [None] * NBUF

    def start_gather(j):
        b = j % NBUF
        copies_g[b] = pltpu.async_copy(table_hbm.at[idx_v.at[j]], rows[b], sem_g[b])

    for j in range(LOOKAHEAD):
        start_gather(j)

    for j in range(NCHUNK):
        b = j % NBUF
        nj = j + LOOKAHEAD
        if nj < NCHUNK:
            bn = nj % NBUF
            if copies_w[bn] is not None:
                copies_w[bn].wait()
            start_gather(nj)
        copies_g[b].wait()
        copies_w[b] = pltpu.async_copy(
            rows[b], out_hbm.at[pl.ds(base + j * CHUNK, CHUNK)], sem_w[b]
        )
    for b in range(NBUF):
        if copies_w[b] is not None:
            copies_w[b].wait()


def kernel(inputs, weight):
    packed = _pack_table(weight.T)
    table = packed.reshape(VOCAB, EMBED)
    v = inputs.astype(jnp.int32)
    l = v % TBLK
    u_full = (v - l) + jnp.where(l < HB, 2 * l, 2 * l - (TBLK - 1))
    t = v - FULL_END
    u_tail = FULL_END + jnp.where(t < TAIL // 2, 2 * t, 2 * t - (TAIL - 1))
    u = jnp.where(v < FULL_END, u_full, u_tail)
    ut = u.T
    inter = jnp.stack(
        [ut[:, : BATCH // 2], ut[:, BATCH // 2 :]], axis=-1
    ).reshape(HIST, BATCH)
    idx = inter.reshape(NW, NCHUNK, CHUNK)
    out = _gather(table, idx)
    fmt = _fmt_out(out.reshape(HIST, BATCH // 2, 2 * EMBED))
    return jnp.transpose(fmt, (2, 0, 1))
